# Initial kernel scaffold; baseline (speedup 1.0000x reference)
import jax
import jax.numpy as jnp
from jax.experimental import pallas as pl


def kernel(gsp, gsp_solar_azimuth, gsp_solar_elevation, gsp_time_utc_fourier, gsp_time_utc_fourier_t0, gsp_y_osgb_fourier, gsp_x_osgb_fourier, gsp_id, emb_table, gsp_t0_idx, include_history):
    raise NotImplementedError("write your pallas kernel here")


# speedup vs baseline: 1.5125x; 1.5125x over previous
import functools

import jax
import jax.numpy as jnp
import numpy as np
from jax import lax
from jax.experimental import pallas as pl
from jax.experimental.pallas import tpu as pltpu
from jax.experimental.pallas import tpu_sc as plsc

B, T, FT, FP, V, E = 4096, 50, 8, 8, 1000, 16
ROW = 1 + FP + FP + FT + FT + 1 + 1 + E
NC, NS, L = 2, 16, 16
NW = NC * NS
EX_W = B // NW
C = 8
NCH = EX_W // C

_G, _A, _EL = 0, C * T, 2 * C * T
_TF = 3 * C * T
_Y = _TF + C * T * FT
_X = _Y + C * FP
_T0 = _X + C * FP
_EM = _T0 + C * FT
SRC_W = _EM + C * E
CHUNK_W = C * T * ROW


def _build_idx() -> np.ndarray:
    ar8 = np.arange(8)
    idx = np.empty((C, T, ROW), np.int32)
    for e in range(C):
        for t in range(T):
            idx[e, t, 0] = _G + e * T + t
            idx[e, t, 1:9] = _Y + e * FP + ar8
            idx[e, t, 9:17] = _X + e * FP + ar8
            idx[e, t, 17:25] = _TF + e * T * FT + t * FT + ar8
            idx[e, t, 25:33] = _T0 + e * FT + ar8
            idx[e, t, 33] = _A + e * T + t
            idx[e, t, 34] = _EL + e * T + t
            idx[e, t, 35:51] = _EM + e * E + np.arange(E)
    return idx.reshape(-1)


_IDX = _build_idx()


def _sc_body(idx_hbm, gsp_hbm, az_hbm, el_hbm, tf_hbm, y_hbm, x_hbm, t0_hbm,
             ids_hbm, table_hbm, inc_hbm,
             out_hbm,
             idxb, srcb, outb, tablev, idsb, incb, sem_in, sem_out):
    wid = lax.axis_index("s") * NC + lax.axis_index("c")
    pltpu.sync_copy(idx_hbm, idxb)
    pltpu.sync_copy(inc_hbm, incb)
    pltpu.sync_copy(table_hbm, tablev)
    pltpu.sync_copy(ids_hbm.at[pl.ds(wid * EX_W, EX_W)], idsb)
    incv = incb[...]
    lanes = lax.iota(jnp.int32, L)

    for ch in range(NCH):
        ex0 = wid * EX_W + ch * C
        cps = [
            pltpu.async_copy(gsp_hbm.at[pl.ds(ex0 * T, C * T)],
                             srcb.at[pl.ds(_G, C * T)], sem_in),
            pltpu.async_copy(az_hbm.at[pl.ds(ex0 * T, C * T)],
                             srcb.at[pl.ds(_A, C * T)], sem_in),
            pltpu.async_copy(el_hbm.at[pl.ds(ex0 * T, C * T)],
                             srcb.at[pl.ds(_EL, C * T)], sem_in),
            pltpu.async_copy(tf_hbm.at[pl.ds(ex0 * T * FT, C * T * FT)],
                             srcb.at[pl.ds(_TF, C * T * FT)], sem_in),
            pltpu.async_copy(y_hbm.at[pl.ds(ex0 * FP, C * FP)],
                             srcb.at[pl.ds(_Y, C * FP)], sem_in),
            pltpu.async_copy(x_hbm.at[pl.ds(ex0 * FP, C * FP)],
                             srcb.at[pl.ds(_X, C * FP)], sem_in),
            pltpu.async_copy(t0_hbm.at[pl.ds(ex0 * FT, C * FT)],
                             srcb.at[pl.ds(_T0, C * FT)], sem_in),
        ]
        for cp in cps:
            cp.wait()
        idwin = idsb[pl.ds((ch * C // L) * L, L)]
        for e in range(C):
            sid = idwin[(ch * C + e) % L]
            srcb[pl.ds(_EM + e * E, L)] = plsc.load_gather(
                tablev, [sid * E + lanes])

        def marker_step(i, _):
            v = srcb[pl.ds(i * L, L)]
            srcb[pl.ds(i * L, L)] = 1.0 + incv * v
            return 0

        lax.fori_loop(0, C * T // L, marker_step, 0)

        def gather_step(i, _):
            outb[pl.ds(i * L, L)] = plsc.load_gather(
                srcb, [idxb[pl.ds(i * L, L)]])
            return 0

        lax.fori_loop(0, CHUNK_W // L, gather_step, 0)

        pltpu.async_copy(outb, out_hbm.at[pl.ds(ex0 * T * ROW, CHUNK_W)],
                         sem_out).wait()


@functools.cache
def _get_sc_kernel():
    return pl.kernel(
        _sc_body,
        out_type=jax.ShapeDtypeStruct((B * T * ROW,), jnp.float32),
        mesh=plsc.VectorSubcoreMesh(core_axis_name="c", subcore_axis_name="s"),
        compiler_params=pltpu.CompilerParams(needs_layout_passes=False),
        scratch_types=[
            pltpu.VMEM((CHUNK_W,), jnp.int32),
            pltpu.VMEM((SRC_W,), jnp.float32),
            pltpu.VMEM((CHUNK_W,), jnp.float32),
            pltpu.VMEM((V * E,), jnp.float32),
            pltpu.VMEM((EX_W,), jnp.int32),
            pltpu.VMEM((L,), jnp.float32),
            pltpu.SemaphoreType.DMA,
            pltpu.SemaphoreType.DMA,
        ],
    )


def kernel(gsp, gsp_solar_azimuth, gsp_solar_elevation, gsp_time_utc_fourier,
           gsp_time_utc_fourier_t0, gsp_y_osgb_fourier, gsp_x_osgb_fourier,
           gsp_id, emb_table, gsp_t0_idx, include_history):
    del gsp_t0_idx
    ids = jnp.clip(gsp_id.reshape(B), 0, V - 1).astype(jnp.int32)
    inc = jnp.full((L,), jnp.asarray(include_history, jnp.float32))
    out = _get_sc_kernel()(
        jnp.asarray(_IDX),
        gsp.reshape(-1),
        gsp_solar_azimuth.reshape(-1),
        gsp_solar_elevation.reshape(-1),
        gsp_time_utc_fourier.reshape(-1),
        gsp_y_osgb_fourier.reshape(-1),
        gsp_x_osgb_fourier.reshape(-1),
        gsp_time_utc_fourier_t0.reshape(-1),
        ids,
        emb_table.reshape(-1),
        inc,
    )
    return out.reshape(B * T, 1, ROW)


# speedup vs baseline: 2.0995x; 1.3881x over previous
import functools

import jax
import jax.numpy as jnp
import numpy as np
from jax import lax
from jax.experimental import pallas as pl
from jax.experimental.pallas import tpu as pltpu
from jax.experimental.pallas import tpu_sc as plsc

B, T, FT, FP, V, E = 4096, 50, 8, 8, 1000, 16
ROW = 1 + FP + FP + FT + FT + 1 + 1 + E
NC, NS, L = 2, 16, 16
NW = NC * NS
EX_W = B // NW
C = 8
NCH = EX_W // C

_G, _A, _EL = 0, C * T, 2 * C * T
_TF = 3 * C * T
_Y = _TF + C * T * FT
_X = _Y + C * FP
_T0 = _X + C * FP
_EM = _T0 + C * FT
SRC_W = _EM + C * E
CHUNK_W = C * T * ROW


def _build_idx() -> np.ndarray:
    ar8 = np.arange(8)
    idx = np.empty((C, T, ROW), np.int32)
    for e in range(C):
        for t in range(T):
            idx[e, t, 0] = _G + e * T + t
            idx[e, t, 1:9] = _Y + e * FP + ar8
            idx[e, t, 9:17] = _X + e * FP + ar8
            idx[e, t, 17:25] = _TF + e * T * FT + t * FT + ar8
            idx[e, t, 25:33] = _T0 + e * FT + ar8
            idx[e, t, 33] = _A + e * T + t
            idx[e, t, 34] = _EL + e * T + t
            idx[e, t, 35:51] = _EM + e * E + np.arange(E)
    return idx.reshape(-1)


_IDX = _build_idx()


def _sc_body(idx_hbm, gsp_hbm, az_hbm, el_hbm, tf_hbm, y_hbm, x_hbm, t0_hbm,
             ids_hbm, table_hbm, inc_hbm,
             out_hbm,
             idxb, srcb0, srcb1, outb0, outb1, tablev, idsb, incb,
             sem_in0, sem_in1, sem_out0, sem_out1):
    wid = lax.axis_index("s") * NC + lax.axis_index("c")
    srcbs, outbs = (srcb0, srcb1), (outb0, outb1)
    sems_in, sems_out = (sem_in0, sem_in1), (sem_out0, sem_out1)

    def issue_inputs(ch, srcb, sem):
        ex0 = wid * EX_W + ch * C
        return [
            pltpu.async_copy(gsp_hbm.at[pl.ds(ex0 * T, C * T)],
                             srcb.at[pl.ds(_G, C * T)], sem),
            pltpu.async_copy(az_hbm.at[pl.ds(ex0 * T, C * T)],
                             srcb.at[pl.ds(_A, C * T)], sem),
            pltpu.async_copy(el_hbm.at[pl.ds(ex0 * T, C * T)],
                             srcb.at[pl.ds(_EL, C * T)], sem),
            pltpu.async_copy(tf_hbm.at[pl.ds(ex0 * T * FT, C * T * FT)],
                             srcb.at[pl.ds(_TF, C * T * FT)], sem),
            pltpu.async_copy(y_hbm.at[pl.ds(ex0 * FP, C * FP)],
                             srcb.at[pl.ds(_Y, C * FP)], sem),
            pltpu.async_copy(x_hbm.at[pl.ds(ex0 * FP, C * FP)],
                             srcb.at[pl.ds(_X, C * FP)], sem),
            pltpu.async_copy(t0_hbm.at[pl.ds(ex0 * FT, C * FT)],
                             srcb.at[pl.ds(_T0, C * FT)], sem),
        ]

    pend_in = [issue_inputs(0, srcb0, sem_in0), None]
    pend_out = [None, None]
    pltpu.sync_copy(idx_hbm, idxb)
    pltpu.sync_copy(inc_hbm, incb)
    pltpu.sync_copy(table_hbm, tablev)
    pltpu.sync_copy(ids_hbm.at[pl.ds(wid * EX_W, EX_W)], idsb)
    incv = incb[...]
    lanes = lax.iota(jnp.int32, L)

    for ch in range(NCH):
        bank = ch % 2
        srcb, outb = srcbs[bank], outbs[bank]
        for cp in pend_in[bank]:
            cp.wait()
        if ch + 1 < NCH:
            pend_in[1 - bank] = issue_inputs(
                ch + 1, srcbs[1 - bank], sems_in[1 - bank])

        idwin = idsb[pl.ds((ch * C // L) * L, L)]
        for e in range(C):
            sid = idwin[(ch * C + e) % L]
            srcb[pl.ds(_EM + e * E, L)] = plsc.load_gather(
                tablev, [sid * E + lanes])

        @plsc.parallel_loop(0, C * T // L, unroll=5)
        def _(i):
            srcb[pl.ds(i * L, L)] = 1.0 + incv * srcb[pl.ds(i * L, L)]

        if pend_out[bank] is not None:
            pend_out[bank].wait()

        @plsc.parallel_loop(0, CHUNK_W // L, unroll=8)
        def _(i):
            outb[pl.ds(i * L, L)] = plsc.load_gather(
                srcb, [idxb[pl.ds(i * L, L)]])

        ex0 = wid * EX_W + ch * C
        pend_out[bank] = pltpu.async_copy(
            outb, out_hbm.at[pl.ds(ex0 * T * ROW, CHUNK_W)], sems_out[bank])

    pend_out[0].wait()
    pend_out[1].wait()


@functools.cache
def _get_sc_kernel():
    return pl.kernel(
        _sc_body,
        out_type=jax.ShapeDtypeStruct((B * T * ROW,), jnp.float32),
        mesh=plsc.VectorSubcoreMesh(core_axis_name="c", subcore_axis_name="s"),
        compiler_params=pltpu.CompilerParams(needs_layout_passes=False),
        scratch_types=[
            pltpu.VMEM((CHUNK_W,), jnp.int32),
            pltpu.VMEM((SRC_W,), jnp.float32),
            pltpu.VMEM((SRC_W,), jnp.float32),
            pltpu.VMEM((CHUNK_W,), jnp.float32),
            pltpu.VMEM((CHUNK_W,), jnp.float32),
            pltpu.VMEM((V * E,), jnp.float32),
            pltpu.VMEM((EX_W,), jnp.int32),
            pltpu.VMEM((L,), jnp.float32),
            pltpu.SemaphoreType.DMA,
            pltpu.SemaphoreType.DMA,
            pltpu.SemaphoreType.DMA,
            pltpu.SemaphoreType.DMA,
        ],
    )


def kernel(gsp, gsp_solar_azimuth, gsp_solar_elevation, gsp_time_utc_fourier,
           gsp_time_utc_fourier_t0, gsp_y_osgb_fourier, gsp_x_osgb_fourier,
           gsp_id, emb_table, gsp_t0_idx, include_history):
    del gsp_t0_idx
    ids = jnp.clip(gsp_id.reshape(B), 0, V - 1).astype(jnp.int32)
    inc = jnp.full((L,), jnp.asarray(include_history, jnp.float32))
    out = _get_sc_kernel()(
        jnp.asarray(_IDX),
        gsp.reshape(-1),
        gsp_solar_azimuth.reshape(-1),
        gsp_solar_elevation.reshape(-1),
        gsp_time_utc_fourier.reshape(-1),
        gsp_y_osgb_fourier.reshape(-1),
        gsp_x_osgb_fourier.reshape(-1),
        gsp_time_utc_fourier_t0.reshape(-1),
        ids,
        emb_table.reshape(-1),
        inc,
    )
    return out.reshape(B * T, 1, ROW)


# speedup vs baseline: 7.9585x; 3.7907x over previous
import functools

import jax
import jax.numpy as jnp
import numpy as np
from jax import lax
from jax.experimental import pallas as pl
from jax.experimental.pallas import tpu as pltpu
from jax.experimental.pallas import tpu_sc as plsc

B, T, FT, FP, V, E = 4096, 50, 8, 8, 1000, 16
ROW = 1 + FP + FP + FT + FT + 1 + 1 + E
VP = 1024
NC, NS, L = 2, 16, 16
NW = NC * NS
EX_W = B // NW
COL_W = EX_W * T

_TT = np.arange(COL_W, dtype=np.int32) % T
_TE = np.arange(COL_W, dtype=np.int32) // T


def _sc_body(tt_hbm, te_hbm, gsp_hbm, az_hbm, el_hbm, tf_hbm, y_hbm, x_hbm,
             t0_hbm, ids_hbm, tab_hbm, inc_hbm,
             out_hbm,
             ttv, tev, gspv, azv, elv, tfv, yv, xv, t0v, idsv, tabv, embl,
             colb0, colb1, incb, sem_in, sem_out0, sem_out1):
    wid = lax.axis_index("s") * NC + lax.axis_index("c")
    b0 = wid * EX_W
    lanes = lax.iota(jnp.int32, L)
    colbs = (colb0, colb1)
    sems = (sem_out0, sem_out1)

    cps = [
        pltpu.async_copy(tt_hbm, ttv, sem_in),
        pltpu.async_copy(te_hbm, tev, sem_in),
        pltpu.async_copy(gsp_hbm.at[:, pl.ds(b0, EX_W)], gspv, sem_in),
        pltpu.async_copy(az_hbm.at[:, pl.ds(b0, EX_W)], azv, sem_in),
        pltpu.async_copy(el_hbm.at[:, pl.ds(b0, EX_W)], elv, sem_in),
        pltpu.async_copy(tf_hbm.at[:, :, pl.ds(b0, EX_W)], tfv, sem_in),
        pltpu.async_copy(y_hbm.at[:, pl.ds(b0, EX_W)], yv, sem_in),
        pltpu.async_copy(x_hbm.at[:, pl.ds(b0, EX_W)], xv, sem_in),
        pltpu.async_copy(t0_hbm.at[:, pl.ds(b0, EX_W)], t0v, sem_in),
        pltpu.async_copy(ids_hbm.at[pl.ds(b0, EX_W)], idsv, sem_in),
        pltpu.async_copy(tab_hbm, tabv, sem_in),
        pltpu.async_copy(inc_hbm, incb, sem_in),
    ]
    for cp in cps:
        cp.wait()
    incv = incb[...]

    def embl_c(c, _):
        def embl_g(g, _):
            ev = lanes + g * L
            idv = plsc.load_gather(idsv, [ev])
            row = plsc.load_gather(tabv, [jnp.full((L,), c, jnp.int32), idv])
            embl[pl.ds(c * EX_W + g * L, L)] = row
            return 0
        return lax.fori_loop(0, EX_W // L, embl_g, 0)

    lax.fori_loop(0, E, embl_c, 0)

    def drain(p):
        pltpu.make_async_copy(
            out_hbm.at[pl.ds(0, COL_W)], colbs[p], sems[p]).wait()

    def emit(colb, p, c):
        pltpu.async_copy(
            colb, out_hbm.at[pl.ds(c * (B * T) + wid * COL_W, COL_W)],
            sems[p])

    def fill_transpose(colb, src, marker):
        @plsc.parallel_loop(0, COL_W // L, unroll=8)
        def _(i):
            tv = ttv[pl.ds(i * L, L)]
            ev = tev[pl.ds(i * L, L)]
            v = plsc.load_gather(src, [tv, ev])
            if marker:
                v = 1.0 + incv * v
            colb[pl.ds(i * L, L)] = v

    def fill_bcast(colb, src2, j, flat_base=None):
        def body(g, _):
            if flat_base is None:
                vv = plsc.load_gather(
                    src2, [jnp.full((L,), j, jnp.int32), lanes + g * L])
            else:
                vv = plsc.load_gather(src2, [flat_base + lanes + g * L])
            for l in range(L):
                sp = jnp.full((L,), vv[l], jnp.float32)
                base = (g * L + l) * T
                colb[pl.ds(base, L)] = sp
                colb[pl.ds(base + L, L)] = sp
                colb[pl.ds(base + 2 * L, L)] = sp
                colb[pl.ds(base + T - L, L)] = sp
            return 0
        lax.fori_loop(0, EX_W // L, body, 0)

    def fill_tf(colb, j):
        @plsc.parallel_loop(0, COL_W // L, unroll=8)
        def _(i):
            tv = ttv[pl.ds(i * L, L)]
            ev = tev[pl.ds(i * L, L)]
            jv = jnp.full((L,), j, jnp.int32)
            colb[pl.ds(i * L, L)] = plsc.load_gather(tfv, [tv, jv, ev])

    fill_transpose(colb0, azv, False)
    emit(colb0, 0, 33)
    fill_transpose(colb1, elv, False)
    emit(colb1, 1, 34)

    drain(0)
    fill_transpose(colb0, gspv, True)
    emit(colb0, 0, 0)

    def pair_group(src2, col0, npair, off):
        def body(jj, _):
            j0 = off + 2 * jj
            c0 = col0 + 2 * jj
            drain(1)
            fill_bcast(colb1, src2, j0)
            emit(colb1, 1, c0)
            drain(0)
            fill_bcast(colb0, src2, j0 + 1)
            emit(colb0, 0, c0 + 1)
            return 0
        lax.fori_loop(0, npair, body, 0)

    pair_group(yv, 1, FP // 2, 0)
    pair_group(xv, 9, FP // 2, 0)

    def tf_pairs(jj, _):
        j0 = 2 * jj
        drain(1)
        fill_tf(colb1, j0)
        emit(colb1, 1, 17 + j0)
        drain(0)
        fill_tf(colb0, j0 + 1)
        emit(colb0, 0, 18 + j0)
        return 0

    lax.fori_loop(0, FT // 2, tf_pairs, 0)

    pair_group(t0v, 25, FT // 2, 0)

    def emb_pairs(jj, _):
        j0 = 2 * jj
        drain(1)
        fill_bcast(colb1, embl, None, flat_base=j0 * EX_W)
        emit(colb1, 1, 35 + j0)
        drain(0)
        fill_bcast(colb0, embl, None, flat_base=(j0 + 1) * EX_W)
        emit(colb0, 0, 36 + j0)
        return 0

    lax.fori_loop(0, E // 2, emb_pairs, 0)

    drain(0)
    drain(1)


@functools.cache
def _get_sc_kernel():
    return pl.kernel(
        _sc_body,
        out_type=jax.ShapeDtypeStruct((B * T * ROW,), jnp.float32),
        mesh=plsc.VectorSubcoreMesh(core_axis_name="c", subcore_axis_name="s"),
        compiler_params=pltpu.CompilerParams(needs_layout_passes=False),
        scratch_types=[
            pltpu.VMEM((COL_W,), jnp.int32),
            pltpu.VMEM((COL_W,), jnp.int32),
            pltpu.VMEM((T, EX_W), jnp.float32),
            pltpu.VMEM((T, EX_W), jnp.float32),
            pltpu.VMEM((T, EX_W), jnp.float32),
            pltpu.VMEM((T, FT, EX_W), jnp.float32),
            pltpu.VMEM((FP, EX_W), jnp.float32),
            pltpu.VMEM((FP, EX_W), jnp.float32),
            pltpu.VMEM((FT, EX_W), jnp.float32),
            pltpu.VMEM((EX_W,), jnp.int32),
            pltpu.VMEM((E, VP), jnp.float32),
            pltpu.VMEM((E * EX_W,), jnp.float32),
            pltpu.VMEM((COL_W,), jnp.float32),
            pltpu.VMEM((COL_W,), jnp.float32),
            pltpu.VMEM((L,), jnp.float32),
            pltpu.SemaphoreType.DMA,
            pltpu.SemaphoreType.DMA,
            pltpu.SemaphoreType.DMA,
        ],
    )


def kernel(gsp, gsp_solar_azimuth, gsp_solar_elevation, gsp_time_utc_fourier,
           gsp_time_utc_fourier_t0, gsp_y_osgb_fourier, gsp_x_osgb_fourier,
           gsp_id, emb_table, gsp_t0_idx, include_history):
    del gsp_t0_idx
    ids = jnp.clip(gsp_id.reshape(B), 0, V - 1).astype(jnp.int32)
    inc = jnp.full((L,), jnp.asarray(include_history, jnp.float32))
    tab = jnp.pad(emb_table.T, ((0, 0), (0, VP - V)))
    out = _get_sc_kernel()(
        jnp.asarray(_TT),
        jnp.asarray(_TE),
        gsp.T,
        gsp_solar_azimuth.T,
        gsp_solar_elevation.T,
        gsp_time_utc_fourier.transpose(1, 2, 0),
        gsp_y_osgb_fourier.reshape(B, FP).T,
        gsp_x_osgb_fourier.reshape(B, FP).T,
        gsp_time_utc_fourier_t0.T,
        ids,
        tab,
        inc,
    )
    return out.reshape(ROW, 1, B * T).transpose(2, 1, 0)


# speedup vs baseline: 8.0622x; 1.0130x over previous
import functools

import jax
import jax.numpy as jnp
import numpy as np
from jax import lax
from jax.experimental import pallas as pl
from jax.experimental.pallas import tpu as pltpu
from jax.experimental.pallas import tpu_sc as plsc

B, T, FT, FP, V, E = 4096, 50, 8, 8, 1000, 16
ROW = 1 + FP + FP + FT + FT + 1 + 1 + E
VP = 1024
NC, NS, L = 2, 16, 16
NW = NC * NS
EX_W = B // NW
COL_W = EX_W * T

_TT = np.arange(COL_W, dtype=np.int32) % T
_TE = np.arange(COL_W, dtype=np.int32) // T


def _sc_body(tt_hbm, te_hbm, gsp_hbm, az_hbm, el_hbm, tf_hbm, y_hbm, x_hbm,
             t0_hbm, ids_hbm, tab_hbm, inc_hbm,
             out_hbm,
             ttv, tev, gspv, azv, elv, tfv, yv, xv, t0v, idsv, tabv, embl,
             colb0, colb1, incb, sem_in, sem_out0, sem_out1):
    wid = lax.axis_index("s") * NC + lax.axis_index("c")
    b0 = wid * EX_W
    lanes = lax.iota(jnp.int32, L)
    colbs = (colb0, colb1)
    sems = (sem_out0, sem_out1)

    cps = [
        pltpu.async_copy(tt_hbm, ttv, sem_in),
        pltpu.async_copy(te_hbm, tev, sem_in),
        pltpu.async_copy(gsp_hbm.at[:, pl.ds(b0, EX_W)], gspv, sem_in),
        pltpu.async_copy(az_hbm.at[:, pl.ds(b0, EX_W)], azv, sem_in),
        pltpu.async_copy(el_hbm.at[:, pl.ds(b0, EX_W)], elv, sem_in),
        pltpu.async_copy(tf_hbm.at[:, :, pl.ds(b0, EX_W)], tfv, sem_in),
        pltpu.async_copy(y_hbm.at[:, pl.ds(b0, EX_W)], yv, sem_in),
        pltpu.async_copy(x_hbm.at[:, pl.ds(b0, EX_W)], xv, sem_in),
        pltpu.async_copy(t0_hbm.at[:, pl.ds(b0, EX_W)], t0v, sem_in),
        pltpu.async_copy(ids_hbm.at[pl.ds(b0, EX_W)], idsv, sem_in),
        pltpu.async_copy(tab_hbm, tabv, sem_in),
        pltpu.async_copy(inc_hbm, incb, sem_in),
    ]
    for cp in cps:
        cp.wait()
    incv = incb[...]

    def embl_c(c, _):
        @plsc.parallel_loop(0, EX_W // L, unroll=4)
        def _(g):
            ev = lanes + g * L
            idv = plsc.load_gather(idsv, [ev])
            row = plsc.load_gather(tabv, [jnp.full((L,), c, jnp.int32), idv])
            embl[pl.ds(c * EX_W + g * L, L)] = row
        return 0

    lax.fori_loop(0, E, embl_c, 0)

    def drain(p):
        pltpu.make_async_copy(
            out_hbm.at[pl.ds(0, COL_W)], colbs[p], sems[p]).wait()

    def emit(colb, p, c):
        pltpu.async_copy(
            colb, out_hbm.at[pl.ds(c * (B * T) + wid * COL_W, COL_W)],
            sems[p])

    def fill_transpose(colb, src, marker):
        @plsc.parallel_loop(0, COL_W // L, unroll=8)
        def _(i):
            tv = ttv[pl.ds(i * L, L)]
            ev = tev[pl.ds(i * L, L)]
            v = plsc.load_gather(src, [tv, ev])
            if marker:
                v = 1.0 + incv * v
            colb[pl.ds(i * L, L)] = v

    def fill_bcast(colb, src2, j, flat_base=None):
        @plsc.parallel_loop(0, EX_W // L, unroll=2)
        def _(g):
            if flat_base is None:
                vv = plsc.load_gather(
                    src2, [jnp.full((L,), j, jnp.int32), lanes + g * L])
            else:
                vv = plsc.load_gather(src2, [flat_base + lanes + g * L])
            for l in range(L):
                sp = jnp.full((L,), vv[l], jnp.float32)
                base = (g * L + l) * T
                colb[pl.ds(base, L)] = sp
                colb[pl.ds(base + L, L)] = sp
                colb[pl.ds(base + 2 * L, L)] = sp
                colb[pl.ds(base + T - L, L)] = sp

    def fill_tf(colb, j):
        @plsc.parallel_loop(0, COL_W // L, unroll=8)
        def _(i):
            tv = ttv[pl.ds(i * L, L)]
            ev = tev[pl.ds(i * L, L)]
            jv = jnp.full((L,), j, jnp.int32)
            colb[pl.ds(i * L, L)] = plsc.load_gather(tfv, [tv, jv, ev])

    fill_transpose(colb0, azv, False)
    emit(colb0, 0, 33)
    fill_transpose(colb1, elv, False)
    emit(colb1, 1, 34)

    drain(0)
    fill_transpose(colb0, gspv, True)
    emit(colb0, 0, 0)

    def pair_group(src2, col0, npair, off):
        def body(jj, _):
            j0 = off + 2 * jj
            c0 = col0 + 2 * jj
            drain(1)
            fill_bcast(colb1, src2, j0)
            emit(colb1, 1, c0)
            drain(0)
            fill_bcast(colb0, src2, j0 + 1)
            emit(colb0, 0, c0 + 1)
            return 0
        lax.fori_loop(0, npair, body, 0)

    pair_group(yv, 1, FP // 2, 0)
    pair_group(xv, 9, FP // 2, 0)

    def tf_pairs(jj, _):
        j0 = 2 * jj
        drain(1)
        fill_tf(colb1, j0)
        emit(colb1, 1, 17 + j0)
        drain(0)
        fill_tf(colb0, j0 + 1)
        emit(colb0, 0, 18 + j0)
        return 0

    lax.fori_loop(0, FT // 2, tf_pairs, 0)

    pair_group(t0v, 25, FT // 2, 0)

    def emb_pairs(jj, _):
        j0 = 2 * jj
        drain(1)
        fill_bcast(colb1, embl, None, flat_base=j0 * EX_W)
        emit(colb1, 1, 35 + j0)
        drain(0)
        fill_bcast(colb0, embl, None, flat_base=(j0 + 1) * EX_W)
        emit(colb0, 0, 36 + j0)
        return 0

    lax.fori_loop(0, E // 2, emb_pairs, 0)

    drain(0)
    drain(1)


@functools.cache
def _get_sc_kernel():
    return pl.kernel(
        _sc_body,
        out_type=jax.ShapeDtypeStruct((B * T * ROW,), jnp.float32),
        mesh=plsc.VectorSubcoreMesh(core_axis_name="c", subcore_axis_name="s"),
        compiler_params=pltpu.CompilerParams(needs_layout_passes=False),
        scratch_types=[
            pltpu.VMEM((COL_W,), jnp.int32),
            pltpu.VMEM((COL_W,), jnp.int32),
            pltpu.VMEM((T, EX_W), jnp.float32),
            pltpu.VMEM((T, EX_W), jnp.float32),
            pltpu.VMEM((T, EX_W), jnp.float32),
            pltpu.VMEM((T, FT, EX_W), jnp.float32),
            pltpu.VMEM((FP, EX_W), jnp.float32),
            pltpu.VMEM((FP, EX_W), jnp.float32),
            pltpu.VMEM((FT, EX_W), jnp.float32),
            pltpu.VMEM((EX_W,), jnp.int32),
            pltpu.VMEM((E, VP), jnp.float32),
            pltpu.VMEM((E * EX_W,), jnp.float32),
            pltpu.VMEM((COL_W,), jnp.float32),
            pltpu.VMEM((COL_W,), jnp.float32),
            pltpu.VMEM((L,), jnp.float32),
            pltpu.SemaphoreType.DMA,
            pltpu.SemaphoreType.DMA,
            pltpu.SemaphoreType.DMA,
        ],
    )


def kernel(gsp, gsp_solar_azimuth, gsp_solar_elevation, gsp_time_utc_fourier,
           gsp_time_utc_fourier_t0, gsp_y_osgb_fourier, gsp_x_osgb_fourier,
           gsp_id, emb_table, gsp_t0_idx, include_history):
    del gsp_t0_idx
    ids = jnp.clip(gsp_id.reshape(B), 0, V - 1).astype(jnp.int32)
    inc = jnp.full((L,), jnp.asarray(include_history, jnp.float32))
    tab = jnp.pad(emb_table.T, ((0, 0), (0, VP - V)))
    out = _get_sc_kernel()(
        jnp.asarray(_TT),
        jnp.asarray(_TE),
        gsp.T,
        gsp_solar_azimuth.T,
        gsp_solar_elevation.T,
        gsp_time_utc_fourier.transpose(1, 2, 0),
        gsp_y_osgb_fourier.reshape(B, FP).T,
        gsp_x_osgb_fourier.reshape(B, FP).T,
        gsp_time_utc_fourier_t0.T,
        ids,
        tab,
        inc,
    )
    return out.reshape(ROW, 1, B * T).transpose(2, 1, 0)


# speedup vs baseline: 12.0549x; 1.4952x over previous
import functools

import jax
import jax.numpy as jnp
import numpy as np
from jax import lax
from jax.experimental import pallas as pl
from jax.experimental.pallas import tpu as pltpu
from jax.experimental.pallas import tpu_sc as plsc

B, T, FT, FP, V, E = 4096, 50, 8, 8, 1000, 16
ROW = 1 + FP + FP + FT + FT + 1 + 1 + E
VP = 1024
NC, NS, L = 2, 16, 16
NW = NC * NS
EX_W = B // NW
COL_W = EX_W * T

def _sc_body(gsp_hbm, az_hbm, el_hbm, tf_hbm, y_hbm, x_hbm,
             t0_hbm, ids_hbm, tab_hbm, inc_hbm,
             out_hbm,
             gspv, azv, elv, tfv, yv, xv, t0v, idsv, tabv, embl,
             colb0, colb1, incb, sem_in, sem_out0, sem_out1):
    wid = lax.axis_index("s") * NC + lax.axis_index("c")
    b0 = wid * EX_W
    lanes = lax.iota(jnp.int32, L)
    colbs = (colb0, colb1)
    sems = (sem_out0, sem_out1)

    cps = [
        pltpu.async_copy(gsp_hbm.at[:, pl.ds(b0, EX_W)], gspv, sem_in),
        pltpu.async_copy(az_hbm.at[:, pl.ds(b0, EX_W)], azv, sem_in),
        pltpu.async_copy(el_hbm.at[:, pl.ds(b0, EX_W)], elv, sem_in),
        pltpu.async_copy(tf_hbm.at[:, :, pl.ds(b0, EX_W)], tfv, sem_in),
        pltpu.async_copy(y_hbm.at[:, pl.ds(b0, EX_W)], yv, sem_in),
        pltpu.async_copy(x_hbm.at[:, pl.ds(b0, EX_W)], xv, sem_in),
        pltpu.async_copy(t0_hbm.at[:, pl.ds(b0, EX_W)], t0v, sem_in),
        pltpu.async_copy(ids_hbm.at[pl.ds(b0, EX_W)], idsv, sem_in),
        pltpu.async_copy(tab_hbm, tabv, sem_in),
        pltpu.async_copy(inc_hbm, incb, sem_in),
    ]
    for cp in cps:
        cp.wait()
    incv = incb[...]

    def embl_c(c, _):
        @plsc.parallel_loop(0, EX_W // L, unroll=4)
        def _(g):
            ev = lanes + g * L
            idv = plsc.load_gather(idsv, [ev])
            row = plsc.load_gather(tabv, [jnp.full((L,), c, jnp.int32), idv])
            embl[pl.ds(c * EX_W + g * L, L)] = row
        return 0

    lax.fori_loop(0, E, embl_c, 0)

    def drain(p):
        pltpu.make_async_copy(
            out_hbm.at[pl.ds(0, COL_W)], colbs[p], sems[p]).wait()

    def emit(colb, p, c):
        pltpu.async_copy(
            colb, out_hbm.at[pl.ds(c * (B * T) + wid * COL_W, COL_W)],
            sems[p])

    def fill_transpose(colb, src, marker):
        def trow(t, _):
            @plsc.parallel_loop(0, EX_W // L, unroll=4)
            def _(g):
                ev = lanes + g * L
                v = plsc.load_gather(src, [jnp.full((L,), t, jnp.int32), ev])
                if marker:
                    v = 1.0 + incv * v
                plsc.store_scatter(colb, [ev * T + t], v)
            return 0
        lax.fori_loop(0, T, trow, 0)

    def fill_bcast(colb, src2, j, flat_base=None):
        @plsc.parallel_loop(0, EX_W // L, unroll=2)
        def _(g):
            if flat_base is None:
                vv = plsc.load_gather(
                    src2, [jnp.full((L,), j, jnp.int32), lanes + g * L])
            else:
                vv = plsc.load_gather(src2, [flat_base + lanes + g * L])
            for l in range(L):
                sp = jnp.full((L,), vv[l], jnp.float32)
                base = (g * L + l) * T
                colb[pl.ds(base, L)] = sp
                colb[pl.ds(base + L, L)] = sp
                colb[pl.ds(base + 2 * L, L)] = sp
                colb[pl.ds(base + T - L, L)] = sp

    def fill_tf(colb, j):
        jv = jnp.full((L,), j, jnp.int32)

        def trow(t, _):
            @plsc.parallel_loop(0, EX_W // L, unroll=4)
            def _(g):
                ev = lanes + g * L
                v = plsc.load_gather(tfv, [jnp.full((L,), t, jnp.int32), jv, ev])
                plsc.store_scatter(colb, [ev * T + t], v)
            return 0
        lax.fori_loop(0, T, trow, 0)

    fill_transpose(colb0, azv, False)
    emit(colb0, 0, 33)
    fill_transpose(colb1, elv, False)
    emit(colb1, 1, 34)

    drain(0)
    fill_transpose(colb0, gspv, True)
    emit(colb0, 0, 0)

    def pair_group(src2, col0, npair, off):
        def body(jj, _):
            j0 = off + 2 * jj
            c0 = col0 + 2 * jj
            drain(1)
            fill_bcast(colb1, src2, j0)
            emit(colb1, 1, c0)
            drain(0)
            fill_bcast(colb0, src2, j0 + 1)
            emit(colb0, 0, c0 + 1)
            return 0
        lax.fori_loop(0, npair, body, 0)

    pair_group(yv, 1, FP // 2, 0)
    pair_group(xv, 9, FP // 2, 0)

    def tf_pairs(jj, _):
        j0 = 2 * jj
        drain(1)
        fill_tf(colb1, j0)
        emit(colb1, 1, 17 + j0)
        drain(0)
        fill_tf(colb0, j0 + 1)
        emit(colb0, 0, 18 + j0)
        return 0

    lax.fori_loop(0, FT // 2, tf_pairs, 0)

    pair_group(t0v, 25, FT // 2, 0)

    def emb_pairs(jj, _):
        j0 = 2 * jj
        drain(1)
        fill_bcast(colb1, embl, None, flat_base=j0 * EX_W)
        emit(colb1, 1, 35 + j0)
        drain(0)
        fill_bcast(colb0, embl, None, flat_base=(j0 + 1) * EX_W)
        emit(colb0, 0, 36 + j0)
        return 0

    lax.fori_loop(0, E // 2, emb_pairs, 0)

    drain(0)
    drain(1)


@functools.cache
def _get_sc_kernel():
    return pl.kernel(
        _sc_body,
        out_type=jax.ShapeDtypeStruct((B * T * ROW,), jnp.float32),
        mesh=plsc.VectorSubcoreMesh(core_axis_name="c", subcore_axis_name="s"),
        compiler_params=pltpu.CompilerParams(needs_layout_passes=False),
        scratch_types=[
            pltpu.VMEM((T, EX_W), jnp.float32),
            pltpu.VMEM((T, EX_W), jnp.float32),
            pltpu.VMEM((T, EX_W), jnp.float32),
            pltpu.VMEM((T, FT, EX_W), jnp.float32),
            pltpu.VMEM((FP, EX_W), jnp.float32),
            pltpu.VMEM((FP, EX_W), jnp.float32),
            pltpu.VMEM((FT, EX_W), jnp.float32),
            pltpu.VMEM((EX_W,), jnp.int32),
            pltpu.VMEM((E, VP), jnp.float32),
            pltpu.VMEM((E * EX_W,), jnp.float32),
            pltpu.VMEM((COL_W,), jnp.float32),
            pltpu.VMEM((COL_W,), jnp.float32),
            pltpu.VMEM((L,), jnp.float32),
            pltpu.SemaphoreType.DMA,
            pltpu.SemaphoreType.DMA,
            pltpu.SemaphoreType.DMA,
        ],
    )


def kernel(gsp, gsp_solar_azimuth, gsp_solar_elevation, gsp_time_utc_fourier,
           gsp_time_utc_fourier_t0, gsp_y_osgb_fourier, gsp_x_osgb_fourier,
           gsp_id, emb_table, gsp_t0_idx, include_history):
    del gsp_t0_idx
    ids = jnp.clip(gsp_id.reshape(B), 0, V - 1).astype(jnp.int32)
    inc = jnp.full((L,), jnp.asarray(include_history, jnp.float32))
    tab = jnp.pad(emb_table.T, ((0, 0), (0, VP - V)))
    out = _get_sc_kernel()(
        gsp.T,
        gsp_solar_azimuth.T,
        gsp_solar_elevation.T,
        gsp_time_utc_fourier.transpose(1, 2, 0),
        gsp_y_osgb_fourier.reshape(B, FP).T,
        gsp_x_osgb_fourier.reshape(B, FP).T,
        gsp_time_utc_fourier_t0.T,
        ids,
        tab,
        inc,
    )
    return out.reshape(ROW, 1, B * T).transpose(2, 1, 0)
